# baseline (speedup 1.0000x reference)
import functools

import jax
import jax.numpy as jnp
from jax import lax
from jax.experimental import pallas as pl
from jax.experimental.pallas import tpu as pltpu
from jax.experimental.pallas import tpu_sc as plsc

N_NODES = 5000
N_EDGES = 80000
H_DIM = 128
EB = 128
NBLK = N_EDGES // EB
CAT = 2 * N_EDGES

_info = plsc.get_sparse_core_info()
_NC = _info.num_cores
_NS = _info.num_subcores
_NW = _NC * _NS

G_PER_W = CAT // _NW
G_CHUNK = 1000
_G_NCH = G_PER_W // G_CHUNK


def _make_gather():
    mesh = plsc.VectorSubcoreMesh(core_axis_name="c", subcore_axis_name="s")

    @functools.partial(
        pl.kernel,
        mesh=mesh,
        out_type=jax.ShapeDtypeStruct((CAT, H_DIM), jnp.float32),
        scratch_types=[
            pltpu.VMEM((G_CHUNK,), jnp.int32),
            pltpu.VMEM((G_CHUNK, H_DIM), jnp.float32),
            pltpu.SemaphoreType.DMA,
        ],
    )
    def gather_k(nu_hbm, idx_hbm, out_hbm, idx_v, rows_v, sem):
        wid = lax.axis_index("s") * _NC + lax.axis_index("c")
        base = wid * G_PER_W
        for ci in range(_G_NCH):
            off = pl.multiple_of(base + ci * G_CHUNK, 8)
            pltpu.sync_copy(idx_hbm.at[pl.ds(off, G_CHUNK)], idx_v)
            pltpu.async_copy(nu_hbm.at[idx_v], rows_v, sem).wait()
            pltpu.sync_copy(rows_v, out_hbm.at[pl.ds(off, G_CHUNK)])

    return gather_k


SLAB_ROWS = 20
SLAB_W = SLAB_ROWS * N_NODES
N_SLABS = N_NODES // SLAB_ROWS
MAX_SLABS_PER_W = (N_SLABS + _NW - 1) // _NW
E_CHUNK = 2048
EPAD = 81920
_E_NCH = EPAD // E_CHUNK


def _make_scatter():
    mesh = plsc.VectorSubcoreMesh(core_axis_name="c", subcore_axis_name="s")

    @functools.partial(
        pl.kernel,
        mesh=mesh,
        compiler_params=pltpu.CompilerParams(needs_layout_passes=False),
        out_type=jax.ShapeDtypeStruct((N_NODES * N_NODES,), jnp.float32),
        scratch_types=[
            pltpu.VMEM((SLAB_W,), jnp.float32),
            pltpu.VMEM((E_CHUNK,), jnp.int32),
            pltpu.VMEM((E_CHUNK,), jnp.float32),
        ],
    )
    def scatter_k(flat_hbm, pred_hbm, padj_hbm, slab_v, idx_v, val_v):
        wid = lax.axis_index("s") * _NC + lax.axis_index("c")
        zero16 = jnp.zeros((16,), jnp.float32)

        for j in range(MAX_SLABS_PER_W):
            s = wid + j * _NW

            @pl.when(s < N_SLABS)
            def _slab():
                r0 = s * SLAB_W

                def zero_body(i, _):
                    slab_v[pl.ds(i * 16, 16)] = zero16
                    return _

                lax.fori_loop(0, SLAB_W // 16, zero_body, 0)

                def chunk_body(c, _):
                    off = pl.multiple_of(c * E_CHUNK, 8)
                    pltpu.sync_copy(flat_hbm.at[pl.ds(off, E_CHUNK)], idx_v)
                    pltpu.sync_copy(pred_hbm.at[pl.ds(off, E_CHUNK)], val_v)

                    def lane_body(k, _):
                        fi = idx_v[pl.ds(k * 16, 16)]
                        local = fi - r0
                        m = (local >= 0) & (local < SLAB_W)
                        li = jnp.where(m, local, 0)
                        v = val_v[pl.ds(k * 16, 16)]
                        plsc.addupdate_scatter(slab_v, [li], v, mask=m)
                        return _

                    lax.fori_loop(0, E_CHUNK // 16, lane_body, 0)
                    return _

                lax.fori_loop(0, _E_NCH, chunk_body, 0)
                out_off = pl.multiple_of(s * SLAB_W, 8)
                pltpu.sync_copy(slab_v, padj_hbm.at[pl.ds(out_off, SLAB_W)])

    return scatter_k


def _nu_body(x_ref, weff_ref, beff_ref, nu_ref):
    acc = jnp.broadcast_to(beff_ref[0:1, :], (N_NODES, H_DIM))
    for k in range(4):
        acc = acc + x_ref[:, k:k + 1] * weff_ref[k:k + 1, :]
    nu_ref[...] = acc


def _edge_gate(ea, gs, gd, wd, bd, wt, bt, ww, bw):
    ed = ea[:, 0:1] * wd + bd
    et = ea[:, 1:2] * wt + bt
    edge = jnp.concatenate([ed, et], axis=1)
    gate = jnp.dot(edge, ww, preferred_element_type=jnp.float32) + bw + gs + gd
    return edge, gate


def _stats_body(ea_ref, gs_ref, gd_ref, wd_ref, bd_ref, wt_ref, bt_ref,
                ww_ref, bw_ref, sums_ref):
    @pl.when(pl.program_id(0) == 0)
    def _():
        sums_ref[...] = jnp.zeros_like(sums_ref)

    _, gate = _edge_gate(ea_ref[...], gs_ref[...], gd_ref[...], wd_ref[...],
                         bd_ref[...], wt_ref[...], bt_ref[...], ww_ref[...],
                         bw_ref[...])
    sums_ref[0:1, :] += jnp.sum(gate, axis=0, keepdims=True)
    sums_ref[1:2, :] += jnp.sum(gate * gate, axis=0, keepdims=True)


def _final_body(ea_ref, gs_ref, gd_ref, src_ref, dst_ref, y_ref, sums_ref,
                wd_ref, bd_ref, wt_ref, bt_ref, ww_ref, bw_ref, g_ref, b_ref,
                wc_ref, bc_ref, pred_ref, flat_ref, loss_ref):
    edge, gate = _edge_gate(ea_ref[...], gs_ref[...], gd_ref[...], wd_ref[...],
                            bd_ref[...], wt_ref[...], bt_ref[...], ww_ref[...],
                            bw_ref[...])
    inv_e = jnp.float32(1.0 / N_EDGES)
    mean = sums_ref[0:1, :] * inv_e
    var = sums_ref[1:2, :] * inv_e - mean * mean
    hnorm = (gate - mean) * lax.rsqrt(var + 1e-5) * g_ref[...] + b_ref[...]
    ef = edge + jnp.maximum(hnorm, 0.0)
    z = jnp.sum(ef * wc_ref[...], axis=1, keepdims=True) + bc_ref[0, 0]
    p = jax.nn.sigmoid(z)
    p_row = jnp.reshape(p, (1, 1, EB))
    pred_ref[...] = p_row
    flat_ref[...] = src_ref[...] * N_NODES + dst_ref[...]

    logp = jnp.clip(jnp.log(p_row), -100.0, None)
    log1mp = jnp.clip(jnp.log(1.0 - p_row), -100.0, None)
    y = y_ref[...]
    part = jnp.sum(y * logp + (1.0 - y) * log1mp)

    @pl.when(pl.program_id(0) == 0)
    def _():
        loss_ref[0, 0] = 0.0

    loss_ref[0, 0] += part

    @pl.when(pl.program_id(0) == NBLK - 1)
    def _():
        loss_ref[0, 0] = loss_ref[0, 0] * jnp.float32(-1.0 / N_EDGES)


def kernel(x, edge_attr, edge_index, y, params):
    lp = params['gcn'][-1]
    we, be = params['node_emb']
    wu, bu = lp['edge_U']
    ww, bw = lp['edge_W']
    wd, bd = params['edge_d']
    wt, bt = params['edge_t']
    gam, bet = lp['bn_e']
    wc, bc = params['cls']

    weff = we @ wu
    beff = (be @ wu + bu).reshape(1, H_DIM)

    src = edge_index[0]
    dst = edge_index[1]

    nu = pl.pallas_call(
        _nu_body,
        out_shape=jax.ShapeDtypeStruct((N_NODES, H_DIM), jnp.float32),
    )(x, weff, beff)

    cat_idx = jnp.concatenate([src, dst])
    gcat = _make_gather()(nu, cat_idx)

    bspec = lambda i: (i, 0)
    spec_e = pl.BlockSpec((EB, 2), bspec)
    spec_gs = pl.BlockSpec((EB, H_DIM), bspec)
    spec_gd = pl.BlockSpec((EB, H_DIM), lambda i: (i + NBLK, 0))
    spec_w1 = lambda r: pl.BlockSpec((r, H_DIM), lambda i: (0, 0))
    spec_h = pl.BlockSpec((1, 64), lambda i: (0, 0))

    wd2, bd2 = wd.reshape(1, 64), bd.reshape(1, 64)
    wt2, bt2 = wt.reshape(1, 64), bt.reshape(1, 64)
    bw2 = bw.reshape(1, H_DIM)
    gam2, bet2 = gam.reshape(1, H_DIM), bet.reshape(1, H_DIM)
    wc2 = wc.reshape(1, H_DIM)
    bc2 = bc.reshape(1, 1)

    sums = pl.pallas_call(
        _stats_body,
        grid=(NBLK,),
        in_specs=[spec_e, spec_gs, spec_gd, spec_h, spec_h, spec_h, spec_h,
                  spec_w1(H_DIM), spec_w1(1)],
        out_specs=pl.BlockSpec((8, H_DIM), lambda i: (0, 0)),
        out_shape=jax.ShapeDtypeStruct((8, H_DIM), jnp.float32),
    )(edge_attr, gcat, gcat, wd2, bd2, wt2, bt2, ww, bw2)

    spec_r = pl.BlockSpec((1, 1, EB), lambda i: (i, 0, 0))
    src3 = src.reshape(NBLK, 1, EB)
    dst3 = dst.reshape(NBLK, 1, EB)
    y3 = y.reshape(NBLK, 1, EB)

    pred3, flat3, loss11 = pl.pallas_call(
        _final_body,
        grid=(NBLK,),
        in_specs=[spec_e, spec_gs, spec_gd, spec_r, spec_r, spec_r,
                  pl.BlockSpec((8, H_DIM), lambda i: (0, 0)),
                  spec_h, spec_h, spec_h, spec_h,
                  spec_w1(H_DIM), spec_w1(1), spec_w1(1), spec_w1(1),
                  spec_w1(1), pl.BlockSpec((1, 1), lambda i: (0, 0))],
        out_specs=[spec_r, spec_r,
                   pl.BlockSpec((1, 1), lambda i: (0, 0),
                                memory_space=pltpu.SMEM)],
        out_shape=[jax.ShapeDtypeStruct((NBLK, 1, EB), jnp.float32),
                   jax.ShapeDtypeStruct((NBLK, 1, EB), jnp.int32),
                   jax.ShapeDtypeStruct((1, 1), jnp.float32)],
    )(edge_attr, gcat, gcat, src3, dst3, y3, sums, wd2, bd2, wt2, bt2, ww,
      bw2, gam2, bet2, wc2, bc2)

    pred_flat = jnp.pad(pred3.reshape(N_EDGES), (0, EPAD - N_EDGES))
    flat_idx = jnp.pad(flat3.reshape(N_EDGES), (0, EPAD - N_EDGES))

    padj = _make_scatter()(flat_idx, pred_flat)
    pred_adj = padj.reshape(1, N_NODES, N_NODES)
    return pred_adj, loss11[0, 0]


# speedup vs baseline: 1.6900x; 1.6900x over previous
import functools

import jax
import jax.numpy as jnp
from jax import lax
from jax.experimental import pallas as pl
from jax.experimental.pallas import tpu as pltpu
from jax.experimental.pallas import tpu_sc as plsc

N_NODES = 5000
N_EDGES = 80000
H_DIM = 128
EB = 128
NBLK = N_EDGES // EB
CAT = 2 * N_EDGES

_info = plsc.get_sparse_core_info()
_NC = _info.num_cores
_NS = _info.num_subcores
_NW = _NC * _NS

G_PER_W = CAT // _NW
G_CHUNK = 1000
_G_NCH = G_PER_W // G_CHUNK


def _make_gather():
    mesh = plsc.VectorSubcoreMesh(core_axis_name="c", subcore_axis_name="s")

    @functools.partial(
        pl.kernel,
        mesh=mesh,
        out_type=jax.ShapeDtypeStruct((CAT, H_DIM), jnp.float32),
        scratch_types=[
            pltpu.VMEM((G_CHUNK,), jnp.int32),
            pltpu.VMEM((G_CHUNK, H_DIM), jnp.float32),
            pltpu.SemaphoreType.DMA,
        ],
    )
    def gather_k(nu_hbm, idx_hbm, out_hbm, idx_v, rows_v, sem):
        wid = lax.axis_index("s") * _NC + lax.axis_index("c")
        base = wid * G_PER_W
        for ci in range(_G_NCH):
            off = pl.multiple_of(base + ci * G_CHUNK, 8)
            pltpu.sync_copy(idx_hbm.at[pl.ds(off, G_CHUNK)], idx_v)
            pltpu.async_copy(nu_hbm.at[idx_v], rows_v, sem).wait()
            pltpu.sync_copy(rows_v, out_hbm.at[pl.ds(off, G_CHUNK)])

    return gather_k


SLAB_ROWS = 250
SLAB_W = SLAB_ROWS * N_NODES
SLABS_PER_CORE = (N_NODES // SLAB_ROWS) // _NC
EPAD = 81920
E_STRIP = EPAD // _NS
ZCHUNK = 10000
_NZ = SLAB_W // ZCHUNK
_KMAX = (_NZ + _NS - 1) // _NS


def _make_scatter():
    mesh = plsc.VectorSubcoreMesh(core_axis_name="c", subcore_axis_name="s")

    @functools.partial(
        pl.kernel,
        mesh=mesh,
        compiler_params=pltpu.CompilerParams(needs_layout_passes=False),
        out_type=jax.ShapeDtypeStruct((N_NODES * N_NODES,), jnp.float32),
        scratch_types=[
            pltpu.VMEM((E_STRIP,), jnp.int32),
            pltpu.VMEM((E_STRIP,), jnp.float32),
            pltpu.VMEM((E_STRIP,), jnp.int32),
            pltpu.VMEM((E_STRIP,), jnp.float32),
            pltpu.VMEM((ZCHUNK,), jnp.float32),
            pltpu.VMEM((ZCHUNK,), jnp.float32),
            pltpu.VMEM_SHARED((SLAB_W,), jnp.float32),
        ],
    )
    def scatter_k(flat_hbm, pred_hbm, padj_hbm, idx_all, val_all, lidx, lval,
                  zbuf, rbuf, slab_sh):
        cid = lax.axis_index("c")
        sid = lax.axis_index("s")
        zero16 = jnp.zeros((16,), jnp.float32)

        eoff = pl.multiple_of(sid * E_STRIP, 8)
        pltpu.sync_copy(flat_hbm.at[pl.ds(eoff, E_STRIP)], idx_all)
        pltpu.sync_copy(pred_hbm.at[pl.ds(eoff, E_STRIP)], val_all)

        def zfill_body(i, _):
            zbuf[pl.ds(i * 16, 16)] = zero16
            return _

        lax.fori_loop(0, ZCHUNK // 16, zfill_body, 0)

        for j in range(SLABS_PER_CORE):
            s = cid * SLABS_PER_CORE + j
            goff = pl.multiple_of(s * SLAB_W, 8)

            for k in range(_KMAX):
                c = sid + k * _NS

                @pl.when(c < _NZ)
                def _zero():
                    pltpu.sync_copy(zbuf, slab_sh.at[pl.ds(c * ZCHUNK, ZCHUNK)])

            plsc.subcore_barrier()

            def mask_body(t, _):
                fi = idx_all[pl.ds(t * 16, 16)]
                local = fi - goff
                m = (local >= 0) & (local < SLAB_W)
                lidx[pl.ds(t * 16, 16)] = jnp.where(m, local, fi & 0xFFFFF)
                v = val_all[pl.ds(t * 16, 16)]
                lval[pl.ds(t * 16, 16)] = jnp.where(m, v, 0.0)
                return _

            lax.fori_loop(0, E_STRIP // 16, mask_body, 0)
            pltpu.sync_copy(lval, slab_sh.at[lidx], add=True)
            plsc.subcore_barrier()

            for k in range(_KMAX):
                c = sid + k * _NS

                @pl.when(c < _NZ)
                def _out():
                    coff = c * ZCHUNK
                    pltpu.sync_copy(slab_sh.at[pl.ds(coff, ZCHUNK)], rbuf)
                    pltpu.sync_copy(rbuf,
                                    padj_hbm.at[pl.ds(goff + coff, ZCHUNK)])

            plsc.subcore_barrier()

    return scatter_k


def _nu_body(x_ref, weff_ref, beff_ref, nu_ref):
    acc = jnp.broadcast_to(beff_ref[0:1, :], (N_NODES, H_DIM))
    for k in range(4):
        acc = acc + x_ref[:, k:k + 1] * weff_ref[k:k + 1, :]
    nu_ref[...] = acc


def _edge_gate(ea, gs, gd, wd, bd, wt, bt, ww, bw):
    ed = ea[:, 0:1] * wd + bd
    et = ea[:, 1:2] * wt + bt
    edge = jnp.concatenate([ed, et], axis=1)
    gate = jnp.dot(edge, ww, preferred_element_type=jnp.float32) + bw + gs + gd
    return edge, gate


def _stats_body(ea_ref, gs_ref, gd_ref, wd_ref, bd_ref, wt_ref, bt_ref,
                ww_ref, bw_ref, sums_ref):
    @pl.when(pl.program_id(0) == 0)
    def _():
        sums_ref[...] = jnp.zeros_like(sums_ref)

    _, gate = _edge_gate(ea_ref[...], gs_ref[...], gd_ref[...], wd_ref[...],
                         bd_ref[...], wt_ref[...], bt_ref[...], ww_ref[...],
                         bw_ref[...])
    sums_ref[0:1, :] += jnp.sum(gate, axis=0, keepdims=True)
    sums_ref[1:2, :] += jnp.sum(gate * gate, axis=0, keepdims=True)


def _final_body(ea_ref, gs_ref, gd_ref, src_ref, dst_ref, y_ref, sums_ref,
                wd_ref, bd_ref, wt_ref, bt_ref, ww_ref, bw_ref, g_ref, b_ref,
                wc_ref, bc_ref, pred_ref, flat_ref, loss_ref):
    edge, gate = _edge_gate(ea_ref[...], gs_ref[...], gd_ref[...], wd_ref[...],
                            bd_ref[...], wt_ref[...], bt_ref[...], ww_ref[...],
                            bw_ref[...])
    inv_e = jnp.float32(1.0 / N_EDGES)
    mean = sums_ref[0:1, :] * inv_e
    var = sums_ref[1:2, :] * inv_e - mean * mean
    hnorm = (gate - mean) * lax.rsqrt(var + 1e-5) * g_ref[...] + b_ref[...]
    ef = edge + jnp.maximum(hnorm, 0.0)
    z = jnp.sum(ef * wc_ref[...], axis=1, keepdims=True) + bc_ref[0, 0]
    p = jax.nn.sigmoid(z)
    p_row = jnp.reshape(p, (1, 1, EB))
    pred_ref[...] = p_row
    flat_ref[...] = src_ref[...] * N_NODES + dst_ref[...]

    logp = jnp.clip(jnp.log(p_row), -100.0, None)
    log1mp = jnp.clip(jnp.log(1.0 - p_row), -100.0, None)
    y = y_ref[...]
    part = jnp.sum(y * logp + (1.0 - y) * log1mp)

    @pl.when(pl.program_id(0) == 0)
    def _():
        loss_ref[0, 0] = 0.0

    loss_ref[0, 0] += part

    @pl.when(pl.program_id(0) == NBLK - 1)
    def _():
        loss_ref[0, 0] = loss_ref[0, 0] * jnp.float32(-1.0 / N_EDGES)


def kernel(x, edge_attr, edge_index, y, params):
    lp = params['gcn'][-1]
    we, be = params['node_emb']
    wu, bu = lp['edge_U']
    ww, bw = lp['edge_W']
    wd, bd = params['edge_d']
    wt, bt = params['edge_t']
    gam, bet = lp['bn_e']
    wc, bc = params['cls']

    weff = we @ wu
    beff = (be @ wu + bu).reshape(1, H_DIM)

    src = edge_index[0]
    dst = edge_index[1]

    nu = pl.pallas_call(
        _nu_body,
        out_shape=jax.ShapeDtypeStruct((N_NODES, H_DIM), jnp.float32),
    )(x, weff, beff)

    cat_idx = jnp.concatenate([src, dst])
    gcat = _make_gather()(nu, cat_idx)

    bspec = lambda i: (i, 0)
    spec_e = pl.BlockSpec((EB, 2), bspec)
    spec_gs = pl.BlockSpec((EB, H_DIM), bspec)
    spec_gd = pl.BlockSpec((EB, H_DIM), lambda i: (i + NBLK, 0))
    spec_w1 = lambda r: pl.BlockSpec((r, H_DIM), lambda i: (0, 0))
    spec_h = pl.BlockSpec((1, 64), lambda i: (0, 0))

    wd2, bd2 = wd.reshape(1, 64), bd.reshape(1, 64)
    wt2, bt2 = wt.reshape(1, 64), bt.reshape(1, 64)
    bw2 = bw.reshape(1, H_DIM)
    gam2, bet2 = gam.reshape(1, H_DIM), bet.reshape(1, H_DIM)
    wc2 = wc.reshape(1, H_DIM)
    bc2 = bc.reshape(1, 1)

    sums = pl.pallas_call(
        _stats_body,
        grid=(NBLK,),
        in_specs=[spec_e, spec_gs, spec_gd, spec_h, spec_h, spec_h, spec_h,
                  spec_w1(H_DIM), spec_w1(1)],
        out_specs=pl.BlockSpec((8, H_DIM), lambda i: (0, 0)),
        out_shape=jax.ShapeDtypeStruct((8, H_DIM), jnp.float32),
    )(edge_attr, gcat, gcat, wd2, bd2, wt2, bt2, ww, bw2)

    spec_r = pl.BlockSpec((1, 1, EB), lambda i: (i, 0, 0))
    src3 = src.reshape(NBLK, 1, EB)
    dst3 = dst.reshape(NBLK, 1, EB)
    y3 = y.reshape(NBLK, 1, EB)

    pred3, flat3, loss11 = pl.pallas_call(
        _final_body,
        grid=(NBLK,),
        in_specs=[spec_e, spec_gs, spec_gd, spec_r, spec_r, spec_r,
                  pl.BlockSpec((8, H_DIM), lambda i: (0, 0)),
                  spec_h, spec_h, spec_h, spec_h,
                  spec_w1(H_DIM), spec_w1(1), spec_w1(1), spec_w1(1),
                  spec_w1(1), pl.BlockSpec((1, 1), lambda i: (0, 0))],
        out_specs=[spec_r, spec_r,
                   pl.BlockSpec((1, 1), lambda i: (0, 0),
                                memory_space=pltpu.SMEM)],
        out_shape=[jax.ShapeDtypeStruct((NBLK, 1, EB), jnp.float32),
                   jax.ShapeDtypeStruct((NBLK, 1, EB), jnp.int32),
                   jax.ShapeDtypeStruct((1, 1), jnp.float32)],
    )(edge_attr, gcat, gcat, src3, dst3, y3, sums, wd2, bd2, wt2, bt2, ww,
      bw2, gam2, bet2, wc2, bc2)

    pred_flat = jnp.pad(pred3.reshape(N_EDGES), (0, EPAD - N_EDGES))
    flat_idx = jnp.pad(flat3.reshape(N_EDGES), (0, EPAD - N_EDGES))

    padj = _make_scatter()(flat_idx, pred_flat)
    pred_adj = padj.reshape(1, N_NODES, N_NODES)
    return pred_adj, loss11[0, 0]
